# baseline (speedup 1.0000x reference)
import jax
import jax.numpy as jnp
from jax.experimental import pallas as pl


def _sum_body(in_ref, out_ref):
    out_ref[...] = jnp.sum(in_ref[...], axis=0, keepdims=True)


def kernel(block_mask, data):
    del block_mask
    wide = data.reshape(131072, 128)
    out = pl.pallas_call(
        _sum_body,
        grid=(16,),
        in_specs=[pl.BlockSpec((8192, 128), lambda i: (i, 0))],
        out_specs=pl.BlockSpec((1, 128), lambda i: (0, 0)),
        out_shape=jax.ShapeDtypeStruct((1, 128), data.dtype),
    )(wide)
    return out


# speedup vs baseline: 2.8956x; 2.8956x over previous
import jax
import jax.numpy as jnp
from jax.experimental import pallas as pl


def _panel_body(in_ref, out_ref):
    out_ref[...] = in_ref[...]


def kernel(block_mask, data):
    del block_mask
    dataT = data.T
    return pl.pallas_call(
        _panel_body,
        grid=(128,),
        in_specs=[pl.BlockSpec((32, 4096), lambda i: (0, i))],
        out_specs=pl.BlockSpec((32, 4096), lambda i: (i, 0)),
        out_shape=jax.ShapeDtypeStruct((4096, 4096), data.dtype),
    )(dataT)


# speedup vs baseline: 6.1960x; 2.1398x over previous
import jax
import jax.numpy as jnp
from jax.experimental import pallas as pl


def _panel_body(in_ref, out_ref):
    for j in range(8):
        out_ref[j * 32:(j + 1) * 32, :] = in_ref[:, j * 4096:(j + 1) * 4096]


def kernel(block_mask, data):
    del block_mask
    dataT = data.T
    return pl.pallas_call(
        _panel_body,
        grid=(16,),
        in_specs=[pl.BlockSpec((32, 32768), lambda i: (0, i))],
        out_specs=pl.BlockSpec((256, 4096), lambda i: (i, 0)),
        out_shape=jax.ShapeDtypeStruct((4096, 4096), data.dtype),
    )(dataT)


# speedup vs baseline: 6.4042x; 1.0336x over previous
import jax
import jax.numpy as jnp
from jax.experimental import pallas as pl


def _panel_body(in_ref, out_ref):
    for j in range(16):
        out_ref[j * 32:(j + 1) * 32, :] = in_ref[:, j * 4096:(j + 1) * 4096]


def kernel(block_mask, data):
    del block_mask
    dataT = data.T
    return pl.pallas_call(
        _panel_body,
        grid=(8,),
        in_specs=[pl.BlockSpec((32, 65536), lambda i: (0, i))],
        out_specs=pl.BlockSpec((512, 4096), lambda i: (i, 0)),
        out_shape=jax.ShapeDtypeStruct((4096, 4096), data.dtype),
    )(dataT)
